# baseline (speedup 1.0000x reference)
import jax
import jax.numpy as jnp
from jax.experimental import pallas as pl
from jax.experimental.pallas import tpu as pltpu

B, D, H, W, K = 64, 512, 32, 32, 64
N = H * W


def _netvlad_kernel(x_ref, wt_ref, b_ref, ct_ref, o_ref):
    xb = x_ref[0]
    ft = jax.lax.dot_general(
        xb, wt_ref[...], (((0,), (0,)), ((), ())),
        preferred_element_type=jnp.float32)
    ft = ft + b_ref[...]
    f3 = ft.reshape(H, W, K)
    m = jnp.max(f3, axis=0, keepdims=True)
    e3 = jnp.exp(f3 - m)
    s = jnp.sum(e3, axis=0, keepdims=True)
    a = (e3 / s).reshape(N, K)
    asum = jnp.sum(a, axis=0, keepdims=True)
    vt = jax.lax.dot_general(
        xb, a, (((1,), (0,)), ((), ())),
        preferred_element_type=jnp.float32)
    vt = vt - asum * ct_ref[...]
    ss = jnp.sum(vt * vt, axis=1, keepdims=True)
    y = vt * jax.lax.rsqrt(jnp.maximum(ss, 1e-24))
    o_ref[0] = y


def kernel(x, w, b_conv, c):
    xf = x.reshape(B, D, N)
    wt = w.T
    ct = c.T
    b2 = b_conv.reshape(1, K)
    out = pl.pallas_call(
        _netvlad_kernel,
        grid=(B,),
        in_specs=[
            pl.BlockSpec((1, D, N), lambda i: (i, 0, 0)),
            pl.BlockSpec((D, K), lambda i: (0, 0)),
            pl.BlockSpec((1, K), lambda i: (0, 0)),
            pl.BlockSpec((D, K), lambda i: (0, 0)),
        ],
        out_specs=pl.BlockSpec((1, D, K), lambda i: (i, 0, 0)),
        out_shape=jax.ShapeDtypeStruct((B, D, K), jnp.float32),
        compiler_params=pltpu.CompilerParams(
            dimension_semantics=("parallel",),
        ),
    )(xf, wt, b2, ct)
    return jnp.transpose(out, (1, 2, 0))
